# baseline (speedup 1.0000x reference)
import jax
import jax.numpy as jnp
from jax.experimental import pallas as pl
from jax.experimental.pallas import tpu as pltpu

G = 2
NQ = 4
CS = 4096
D = 128
DG = D // G
BT = 16 * 2048
TB = 512
NB = BT // TB


def _bf(v):
    return v.astype(jnp.bfloat16)


def _distances(r, cbT, cbn):
    ab = jnp.dot(_bf(r), cbT, preferred_element_type=jnp.float32)
    rsum = jnp.sum(r * r, axis=-1, keepdims=True)
    return (rsum - 2.0 * ab) + cbn[None, :]


def _s0_body(x_ref, Win_ref, bin_ref, cbT_ref, cbn_ref,
             xin_ref, idx_ref, dmin_ref):
    g = pl.program_id(0)
    xg = x_ref[0]
    xin = jnp.dot(_bf(xg), _bf(Win_ref[g]),
                  preferred_element_type=jnp.float32) + bin_ref[g][None, :]
    xin_ref[...] = xin[None]
    d = _distances(xin, cbT_ref[g], cbn_ref[g])
    idx_ref[...] = jnp.argmin(d, axis=-1).astype(jnp.int32)[None, None, None, :]
    dmin_ref[...] = jnp.min(d, axis=-1)[None, None, None, :]


def _sq_body(rprev_ref, qprev_ref, cbT_ref, cbn_ref,
             r_ref, idx_ref, dmin_ref):
    g = pl.program_id(0)
    r = rprev_ref[0] - qprev_ref[0]
    r_ref[...] = r[None]
    d = _distances(r, cbT_ref[g], cbn_ref[g])
    idx_ref[...] = jnp.argmin(d, axis=-1).astype(jnp.int32)[None, None, None, :]
    dmin_ref[...] = jnp.min(d, axis=-1)[None, None, None, :]


def _z_body(x_ref, xin_ref, r3_ref, q3_ref, Wout_ref, bout_ref, cm_ref,
            quant_ref, recon_ref, loss_ref):
    x = x_ref[...]
    outs = []
    for g in range(G):
        r4 = r3_ref[g] - q3_ref[g]
        qout = xin_ref[g] - r4
        xout = jnp.dot(_bf(qout), _bf(Wout_ref[g]),
                       preferred_element_type=jnp.float32) + bout_ref[g][None, :]
        outs.append(xout)
    quantized = jnp.concatenate(outs, axis=-1)
    quant_ref[...] = quantized
    diff = x - quantized
    recon = jnp.sum(diff * diff, axis=-1) * (1.0 / D)
    recon_ref[...] = recon
    loss_ref[...] = recon + cm_ref[0, 0]


def _full(spec):
    return pl.BlockSpec(spec, lambda *_: tuple(0 for _ in spec))


_IDX_SPEC = pl.BlockSpec((1, 1, 1, TB), lambda g, i: (g, i, 0, 0))
_R_SPEC = pl.BlockSpec((1, TB, DG), lambda g, i: (g, i, 0))
_CP = pltpu.CompilerParams(
    dimension_semantics=("parallel", "parallel"))


def _stage0(xg, W_in, b_in, cbT0, cbn0, interpret=False):
    return pl.pallas_call(
        _s0_body,
        grid=(G, NB),
        in_specs=[
            _R_SPEC,
            _full((G, DG, DG)),
            _full((G, DG)),
            _full((G, DG, CS)),
            _full((G, CS)),
        ],
        out_specs=[_R_SPEC, _IDX_SPEC, _IDX_SPEC],
        out_shape=[
            jax.ShapeDtypeStruct((G, BT, DG), jnp.float32),
            jax.ShapeDtypeStruct((G, NB, 1, TB), jnp.int32),
            jax.ShapeDtypeStruct((G, NB, 1, TB), jnp.float32),
        ],
        compiler_params=_CP,
        interpret=interpret,
    )(xg, W_in, b_in, cbT0, cbn0)


def _stageq(rprev, qprev, cbTq, cbnq, interpret=False):
    return pl.pallas_call(
        _sq_body,
        grid=(G, NB),
        in_specs=[_R_SPEC, _R_SPEC, _full((G, DG, CS)), _full((G, CS))],
        out_specs=[_R_SPEC, _IDX_SPEC, _IDX_SPEC],
        out_shape=[
            jax.ShapeDtypeStruct((G, BT, DG), jnp.float32),
            jax.ShapeDtypeStruct((G, NB, 1, TB), jnp.int32),
            jax.ShapeDtypeStruct((G, NB, 1, TB), jnp.float32),
        ],
        compiler_params=_CP,
        interpret=interpret,
    )(rprev, qprev, cbTq, cbnq)


def _stagez(xf, xin, r3, q3, W_out, b_out, cm, interpret=False):
    return pl.pallas_call(
        _z_body,
        grid=(NB,),
        in_specs=[
            pl.BlockSpec((TB, D), lambda i: (i, 0)),
            pl.BlockSpec((G, TB, DG), lambda i: (0, i, 0)),
            pl.BlockSpec((G, TB, DG), lambda i: (0, i, 0)),
            pl.BlockSpec((G, TB, DG), lambda i: (0, i, 0)),
            _full((G, DG, DG)),
            _full((G, DG)),
            _full((1, 1)),
        ],
        out_specs=[
            pl.BlockSpec((TB, D), lambda i: (i, 0)),
            pl.BlockSpec((TB,), lambda i: (i,)),
            pl.BlockSpec((TB,), lambda i: (i,)),
        ],
        out_shape=[
            jax.ShapeDtypeStruct((BT, D), jnp.float32),
            jax.ShapeDtypeStruct((BT,), jnp.float32),
            jax.ShapeDtypeStruct((BT,), jnp.float32),
        ],
        compiler_params=pltpu.CompilerParams(
            dimension_semantics=("parallel",)),
        interpret=interpret,
    )(xf, xin, r3, q3, W_out, b_out, cm)


def _gather_rows(cb_q, idx):
    return jnp.take_along_axis(cb_q, idx[:, :, None], axis=1)


def _pipeline(x, W_in, b_in, W_out, b_out, codebooks, interpret=False):
    Bb, Tt, _ = x.shape
    xf = x.reshape(BT, D)
    xg = xf.reshape(BT, G, DG).transpose(1, 0, 2)
    cbT = codebooks.transpose(0, 1, 3, 2).astype(jnp.bfloat16)
    cbn = jnp.sum(codebooks * codebooks, axis=-1)

    xin, idx0, dmin0 = _stage0(xg, W_in, b_in, cbT[:, 0], cbn[:, 0],
                               interpret=interpret)
    idxs = [idx0.reshape(G, BT)]
    dmins = [dmin0.reshape(G, BT)]
    r = xin
    for q in range(1, NQ):
        quant = _gather_rows(codebooks[:, q - 1], idxs[-1])
        r, idxq, dminq = _stageq(r, quant, cbT[:, q], cbn[:, q],
                                 interpret=interpret)
        idxs.append(idxq.reshape(G, BT))
        dmins.append(dminq.reshape(G, BT))
    q3 = _gather_rows(codebooks[:, NQ - 1], idxs[-1])

    cm = jnp.sum(jnp.stack(dmins)) / (G * NQ * BT * DG)
    quantized, recon, loss = _stagez(xf, xin, r, q3, W_out, b_out,
                                     cm.reshape(1, 1), interpret=interpret)
    return (quantized.reshape(Bb, Tt, D), loss.reshape(Bb, Tt),
            cm, recon.reshape(Bb, Tt))


def kernel(x, W_in, b_in, W_out, b_out, codebooks):
    return _pipeline(x, W_in, b_in, W_out, b_out, codebooks)


# speedup vs baseline: 1.2978x; 1.2978x over previous
import jax
import jax.numpy as jnp
from jax.experimental import pallas as pl
from jax.experimental.pallas import tpu as pltpu

G = 2
NQ = 4
CS = 4096
D = 128
DG = D // G
BT = 16 * 2048
TB = 512
NB = BT // TB


def _bf(v):
    return v.astype(jnp.bfloat16)


def _full(spec):
    return pl.BlockSpec(spec, lambda *_: tuple(0 for _ in spec))


def _byte_planes(cb):
    bits = jax.lax.bitcast_convert_type(cb, jnp.int32)
    return [((bits >> (8 * k)) & 255).astype(jnp.bfloat16) for k in range(4)]


def _plane_gather(oh, p0, p1, p2, p3):
    b0 = jnp.dot(oh, p0, preferred_element_type=jnp.float32).astype(jnp.int32)
    b1 = jnp.dot(oh, p1, preferred_element_type=jnp.float32).astype(jnp.int32)
    b2 = jnp.dot(oh, p2, preferred_element_type=jnp.float32).astype(jnp.int32)
    b3 = jnp.dot(oh, p3, preferred_element_type=jnp.float32).astype(jnp.int32)
    bits = b0 | (b1 << 8) | (b2 << 16) | (b3 << 24)
    return jax.lax.bitcast_convert_type(bits, jnp.float32)


def _vq_body(x_ref, Win_ref, bin_ref, Wout_ref, bout_ref,
             cbT_ref, p0_ref, p1_ref, p2_ref, p3_ref, cbn_ref,
             quant_ref, recon_ref, dsum_ref):
    x = x_ref[...]
    iota = jax.lax.broadcasted_iota(jnp.int32, (TB, CS), 1)

    xin = [None] * G
    r = [None] * G
    for g in range(G):
        xg = x[:, g * DG:(g + 1) * DG]
        xin[g] = jnp.dot(_bf(xg), _bf(Win_ref[g]),
                         preferred_element_type=jnp.float32) + bin_ref[g][None, :]
        r[g] = xin[g]

    dsum = jnp.zeros((), jnp.float32)
    qlast = [None] * G
    for q in range(NQ):
        for g in range(G):
            ab = jnp.dot(_bf(r[g]), cbT_ref[g, q],
                         preferred_element_type=jnp.float32)
            rsum = jnp.sum(r[g] * r[g], axis=-1, keepdims=True)
            d = (rsum - 2.0 * ab) + cbn_ref[g, q][None, :]
            idx = jnp.argmin(d, axis=-1)
            dsum = dsum + jnp.sum(jnp.min(d, axis=-1))
            oh = (iota == idx[:, None]).astype(jnp.bfloat16)
            quant = _plane_gather(oh, p0_ref[g, q], p1_ref[g, q],
                                  p2_ref[g, q], p3_ref[g, q])
            if q < NQ - 1:
                r[g] = r[g] - quant
            else:
                qlast[g] = quant

    outs = []
    for g in range(G):
        qout = xin[g] - (r[g] - qlast[g])
        outs.append(jnp.dot(_bf(qout), _bf(Wout_ref[g]),
                            preferred_element_type=jnp.float32)
                    + bout_ref[g][None, :])
    quantized = jnp.concatenate(outs, axis=-1)
    quant_ref[...] = quantized
    diff = x - quantized
    recon_ref[...] = jnp.sum(diff * diff, axis=-1) * (1.0 / D)
    dsum_ref[...] = dsum.reshape(1, 1, 1)


def _loss_body(dsum_ref, recon_ref, loss_ref, cm_ref):
    cm = jnp.sum(dsum_ref[...]) * (1.0 / (G * NQ * BT * DG))
    loss_ref[...] = recon_ref[...] + cm
    cm_ref[...] = cm.reshape(1, 1)


def _pipeline(x, W_in, b_in, W_out, b_out, codebooks, interpret=False):
    Bb, Tt, _ = x.shape
    xf = x.reshape(BT, D)
    cbT = codebooks.transpose(0, 1, 3, 2).astype(jnp.bfloat16)
    planes = _byte_planes(codebooks)
    cbn = jnp.sum(codebooks * codebooks, axis=-1)

    quantized, recon, dsum = pl.pallas_call(
        _vq_body,
        grid=(NB,),
        in_specs=[
            pl.BlockSpec((TB, D), lambda i: (i, 0)),
            _full((G, DG, DG)),
            _full((G, DG)),
            _full((G, DG, DG)),
            _full((G, DG)),
            _full((G, NQ, DG, CS)),
            _full((G, NQ, CS, DG)),
            _full((G, NQ, CS, DG)),
            _full((G, NQ, CS, DG)),
            _full((G, NQ, CS, DG)),
            _full((G, NQ, CS)),
        ],
        out_specs=[
            pl.BlockSpec((TB, D), lambda i: (i, 0)),
            pl.BlockSpec((TB,), lambda i: (i,)),
            pl.BlockSpec((1, 1, 1), lambda i: (i, 0, 0)),
        ],
        out_shape=[
            jax.ShapeDtypeStruct((BT, D), jnp.float32),
            jax.ShapeDtypeStruct((BT,), jnp.float32),
            jax.ShapeDtypeStruct((NB, 1, 1), jnp.float32),
        ],
        compiler_params=pltpu.CompilerParams(
            dimension_semantics=("parallel",)),
        interpret=interpret,
    )(xf, W_in, b_in, W_out, b_out, cbT, *planes, cbn)

    loss, cm = pl.pallas_call(
        _loss_body,
        grid=(NB,),
        in_specs=[
            _full((NB, 1, 1)),
            pl.BlockSpec((TB,), lambda i: (i,)),
        ],
        out_specs=[
            pl.BlockSpec((TB,), lambda i: (i,)),
            pl.BlockSpec((1, 1), lambda i: (0, 0)),
        ],
        out_shape=[
            jax.ShapeDtypeStruct((BT,), jnp.float32),
            jax.ShapeDtypeStruct((1, 1), jnp.float32),
        ],
        compiler_params=pltpu.CompilerParams(
            dimension_semantics=("arbitrary",)),
        interpret=interpret,
    )(dsum, recon)

    return (quantized.reshape(Bb, Tt, D), loss.reshape(Bb, Tt),
            cm.reshape(()), recon.reshape(Bb, Tt))


def kernel(x, W_in, b_in, W_out, b_out, codebooks):
    return _pipeline(x, W_in, b_in, W_out, b_out, codebooks)


# speedup vs baseline: 2.6407x; 2.0348x over previous
import jax
import jax.numpy as jnp
from jax.experimental import pallas as pl
from jax.experimental.pallas import tpu as pltpu

G = 2
NQ = 4
CS = 4096
D = 128
DG = D // G
BT = 16 * 2048
TB = 512
NB = BT // TB


def _bf(v):
    return v.astype(jnp.bfloat16)


def _full(spec):
    return pl.BlockSpec(spec, lambda *_: tuple(0 for _ in spec))


def _byte_planes_cat(cb):
    bits = jax.lax.bitcast_convert_type(cb, jnp.int32)
    planes = [((bits >> (8 * k)) & 255).astype(jnp.bfloat16)
              for k in range(4)]
    return jnp.concatenate(planes, axis=-1)


def _block_diag2(w):
    z = jnp.zeros((DG, DG), w.dtype)
    return jnp.block([[w[0], z], [z, w[1]]])


def _assemble_f32(bytes_f32):
    b = bytes_f32.astype(jnp.int32)
    bits = (b[:, 0 * DG:1 * DG]
            | (b[:, 1 * DG:2 * DG] << 8)
            | (b[:, 2 * DG:3 * DG] << 16)
            | (b[:, 3 * DG:4 * DG] << 24))
    return jax.lax.bitcast_convert_type(bits, jnp.float32)


def _vq_body(x_ref, Winbd_ref, bin_ref, Woutbd_ref, bout_ref,
             cbTbd_ref, pcat_ref, cbn_ref,
             quant_ref, recon_ref, dsum_ref):
    x = x_ref[...]
    iota = jax.lax.broadcasted_iota(jnp.int32, (TB, CS), 1)

    xin = jnp.dot(_bf(x), _bf(Winbd_ref[...]),
                  preferred_element_type=jnp.float32) + bin_ref[...][None, :]
    r = xin

    dsum = jnp.zeros((), jnp.float32)
    qlast = None
    for q in range(NQ):
        ab = jnp.dot(_bf(r), cbTbd_ref[q],
                     preferred_element_type=jnp.float32)
        quants = []
        for g in range(G):
            rg = r[:, g * DG:(g + 1) * DG]
            rsum = jnp.sum(rg * rg, axis=-1, keepdims=True)
            d = (rsum - 2.0 * ab[:, g * CS:(g + 1) * CS]) \
                + cbn_ref[q, g * CS:(g + 1) * CS][None, :]
            idx = jnp.argmin(d, axis=-1)
            dsum = dsum + jnp.sum(jnp.min(d, axis=-1))
            oh = (iota == idx[:, None]).astype(jnp.bfloat16)
            bytes_f32 = jnp.dot(oh, pcat_ref[g, q],
                                preferred_element_type=jnp.float32)
            quants.append(_assemble_f32(bytes_f32))
        quant = jnp.concatenate(quants, axis=-1)
        if q < NQ - 1:
            r = r - quant
        else:
            qlast = quant

    qout = xin - (r - qlast)
    quantized = jnp.dot(_bf(qout), _bf(Woutbd_ref[...]),
                        preferred_element_type=jnp.float32) \
        + bout_ref[...][None, :]
    quant_ref[...] = quantized
    diff = x - quantized
    recon_ref[...] = jnp.sum(diff * diff, axis=-1) * (1.0 / D)
    dsum_ref[...] = dsum.reshape(1, 1, 1)


def _loss_body(dsum_ref, recon_ref, loss_ref, cm_ref):
    cm = jnp.sum(dsum_ref[...]) * (1.0 / (G * NQ * BT * DG))
    loss_ref[...] = recon_ref[...] + cm
    cm_ref[...] = cm.reshape(1, 1)


def _pipeline(x, W_in, b_in, W_out, b_out, codebooks, interpret=False):
    Bb, Tt, _ = x.shape
    xf = x.reshape(BT, D)

    cbT = codebooks.transpose(0, 1, 3, 2).astype(jnp.bfloat16)
    zpad = jnp.zeros((NQ, DG, CS), jnp.bfloat16)
    top = jnp.concatenate([cbT[0], zpad], axis=2)
    bot = jnp.concatenate([zpad, cbT[1]], axis=2)
    cbTbd = jnp.concatenate([top, bot], axis=1)
    pcat = _byte_planes_cat(codebooks)
    cbn = jnp.sum(codebooks * codebooks, axis=-1)
    cbn_cat = jnp.concatenate([cbn[0], cbn[1]], axis=-1)
    Winbd = _block_diag2(W_in)
    Woutbd = _block_diag2(W_out)
    bin_cat = b_in.reshape(D)
    bout_cat = b_out.reshape(D)

    quantized, recon, dsum = pl.pallas_call(
        _vq_body,
        grid=(NB,),
        in_specs=[
            pl.BlockSpec((TB, D), lambda i: (i, 0)),
            _full((D, D)),
            _full((D,)),
            _full((D, D)),
            _full((D,)),
            _full((NQ, D, 2 * CS)),
            _full((G, NQ, CS, 4 * DG)),
            _full((NQ, 2 * CS)),
        ],
        out_specs=[
            pl.BlockSpec((TB, D), lambda i: (i, 0)),
            pl.BlockSpec((TB,), lambda i: (i,)),
            pl.BlockSpec((1, 1, 1), lambda i: (i, 0, 0)),
        ],
        out_shape=[
            jax.ShapeDtypeStruct((BT, D), jnp.float32),
            jax.ShapeDtypeStruct((BT,), jnp.float32),
            jax.ShapeDtypeStruct((NB, 1, 1), jnp.float32),
        ],
        compiler_params=pltpu.CompilerParams(
            dimension_semantics=("parallel",)),
        interpret=interpret,
    )(xf, Winbd, bin_cat, Woutbd, bout_cat, cbTbd, pcat, cbn_cat)

    loss, cm = pl.pallas_call(
        _loss_body,
        grid=(NB,),
        in_specs=[
            _full((NB, 1, 1)),
            pl.BlockSpec((TB,), lambda i: (i,)),
        ],
        out_specs=[
            pl.BlockSpec((TB,), lambda i: (i,)),
            pl.BlockSpec((1, 1), lambda i: (0, 0)),
        ],
        out_shape=[
            jax.ShapeDtypeStruct((BT,), jnp.float32),
            jax.ShapeDtypeStruct((1, 1), jnp.float32),
        ],
        compiler_params=pltpu.CompilerParams(
            dimension_semantics=("arbitrary",)),
        interpret=interpret,
    )(dsum, recon)

    return (quantized.reshape(Bb, Tt, D), loss.reshape(Bb, Tt),
            cm.reshape(()), recon.reshape(Bb, Tt))


def kernel(x, W_in, b_in, W_out, b_out, codebooks):
    return _pipeline(x, W_in, b_in, W_out, b_out, codebooks)


# speedup vs baseline: 2.8528x; 1.0803x over previous
import jax
import jax.numpy as jnp
from jax.experimental import pallas as pl
from jax.experimental.pallas import tpu as pltpu

G = 2
NQ = 4
CS = 4096
D = 128
DG = D // G
BT = 16 * 2048
TB = 512
NB = BT // TB


def _bf(v):
    return v.astype(jnp.bfloat16)


def _full(spec):
    return pl.BlockSpec(spec, lambda *_: tuple(0 for _ in spec))


def _byte_planes_cat(cb):
    bits = jax.lax.bitcast_convert_type(cb, jnp.int32)
    planes = [((bits >> (8 * k)) & 255).astype(jnp.bfloat16)
              for k in range(4)]
    return jnp.concatenate(planes, axis=-1)


def _block_diag2(w):
    z = jnp.zeros((DG, DG), w.dtype)
    return jnp.block([[w[0], z], [z, w[1]]])


def _assemble_f32(bytes_f32):
    b = bytes_f32.astype(jnp.int32)
    bits = (b[:, 0 * DG:1 * DG]
            | (b[:, 1 * DG:2 * DG] << 8)
            | (b[:, 2 * DG:3 * DG] << 16)
            | (b[:, 3 * DG:4 * DG] << 24))
    return jax.lax.bitcast_convert_type(bits, jnp.float32)


def _vq_body(x_ref, Winbd_ref, bin_ref, Woutbd_ref, bout_ref,
             cbTbd_ref, pcat_ref, cbn_ref,
             quant_ref, recon_ref, dsum_ref):
    x = x_ref[...]
    iota = jax.lax.broadcasted_iota(jnp.int32, (TB, CS), 1)

    xin = jnp.dot(_bf(x), _bf(Winbd_ref[...]),
                  preferred_element_type=jnp.float32) + bin_ref[...][None, :]
    r = xin

    dsum = jnp.zeros((), jnp.float32)
    for q in range(NQ):
        ab = jnp.dot(_bf(r), cbTbd_ref[q],
                     preferred_element_type=jnp.float32)
        quants = []
        for g in range(G):
            rg = r[:, g * DG:(g + 1) * DG]
            rsum = jnp.sum(rg * rg, axis=-1, keepdims=True)
            d = (rsum - 2.0 * ab[:, g * CS:(g + 1) * CS]) \
                + cbn_ref[q, g * CS:(g + 1) * CS][None, :]
            idx = jnp.argmin(d, axis=-1)
            oh = (iota == idx[:, None]).astype(jnp.bfloat16)
            bytes_f32 = jnp.dot(oh, pcat_ref[g, q],
                                preferred_element_type=jnp.float32)
            quants.append(_assemble_f32(bytes_f32))
        quant = jnp.concatenate(quants, axis=-1)
        r = r - quant
        dsum = dsum + jnp.sum(r * r)

    qout = xin - r
    quantized = jnp.dot(_bf(qout), _bf(Woutbd_ref[...]),
                        preferred_element_type=jnp.float32) \
        + bout_ref[...][None, :]
    quant_ref[...] = quantized
    diff = x - quantized
    recon_ref[...] = jnp.sum(diff * diff, axis=-1) * (1.0 / D)
    dsum_ref[...] = dsum.reshape(1, 1, 1)


def _loss_body(dsum_ref, recon_ref, loss_ref, cm_ref):
    cm = jnp.sum(dsum_ref[...]) * (1.0 / (G * NQ * BT * DG))
    loss_ref[...] = recon_ref[...] + cm
    cm_ref[...] = cm.reshape(1, 1)


def _pipeline(x, W_in, b_in, W_out, b_out, codebooks, interpret=False):
    Bb, Tt, _ = x.shape
    xf = x.reshape(BT, D)

    cbT = codebooks.transpose(0, 1, 3, 2).astype(jnp.bfloat16)
    zpad = jnp.zeros((NQ, DG, CS), jnp.bfloat16)
    top = jnp.concatenate([cbT[0], zpad], axis=2)
    bot = jnp.concatenate([zpad, cbT[1]], axis=2)
    cbTbd = jnp.concatenate([top, bot], axis=1)
    pcat = _byte_planes_cat(codebooks)
    cbn = jnp.sum(codebooks * codebooks, axis=-1)
    cbn_cat = jnp.concatenate([cbn[0], cbn[1]], axis=-1)
    Winbd = _block_diag2(W_in)
    Woutbd = _block_diag2(W_out)
    bin_cat = b_in.reshape(D)
    bout_cat = b_out.reshape(D)

    quantized, recon, dsum = pl.pallas_call(
        _vq_body,
        grid=(NB,),
        in_specs=[
            pl.BlockSpec((TB, D), lambda i: (i, 0)),
            _full((D, D)),
            _full((D,)),
            _full((D, D)),
            _full((D,)),
            _full((NQ, D, 2 * CS)),
            _full((G, NQ, CS, 4 * DG)),
            _full((NQ, 2 * CS)),
        ],
        out_specs=[
            pl.BlockSpec((TB, D), lambda i: (i, 0)),
            pl.BlockSpec((TB,), lambda i: (i,)),
            pl.BlockSpec((1, 1, 1), lambda i: (i, 0, 0)),
        ],
        out_shape=[
            jax.ShapeDtypeStruct((BT, D), jnp.float32),
            jax.ShapeDtypeStruct((BT,), jnp.float32),
            jax.ShapeDtypeStruct((NB, 1, 1), jnp.float32),
        ],
        compiler_params=pltpu.CompilerParams(
            dimension_semantics=("parallel",)),
        interpret=interpret,
    )(xf, Winbd, bin_cat, Woutbd, bout_cat, cbTbd, pcat, cbn_cat)

    loss, cm = pl.pallas_call(
        _loss_body,
        grid=(NB,),
        in_specs=[
            _full((NB, 1, 1)),
            pl.BlockSpec((TB,), lambda i: (i,)),
        ],
        out_specs=[
            pl.BlockSpec((TB,), lambda i: (i,)),
            pl.BlockSpec((1, 1), lambda i: (0, 0)),
        ],
        out_shape=[
            jax.ShapeDtypeStruct((BT,), jnp.float32),
            jax.ShapeDtypeStruct((1, 1), jnp.float32),
        ],
        compiler_params=pltpu.CompilerParams(
            dimension_semantics=("arbitrary",)),
        interpret=interpret,
    )(dsum, recon)

    return (quantized.reshape(Bb, Tt, D), loss.reshape(Bb, Tt),
            cm.reshape(()), recon.reshape(Bb, Tt))


def kernel(x, W_in, b_in, W_out, b_out, codebooks):
    return _pipeline(x, W_in, b_in, W_out, b_out, codebooks)


# speedup vs baseline: 3.5729x; 1.2524x over previous
import jax
import jax.numpy as jnp
from jax.experimental import pallas as pl
from jax.experimental.pallas import tpu as pltpu

G = 2
NQ = 4
CS = 4096
D = 128
DG = D // G
BT = 16 * 2048
TB = 512
NB = BT // TB


def _bf(v):
    return v.astype(jnp.bfloat16)


def _full(spec):
    return pl.BlockSpec(spec, lambda *_: tuple(0 for _ in spec))


def _byte_planes_cat(cb):
    bits = jax.lax.bitcast_convert_type(cb, jnp.int32)
    planes = [((bits >> (8 * k)) & 255).astype(jnp.bfloat16)
              for k in range(4)]
    return jnp.concatenate(planes, axis=-1)


def _block_diag2(w):
    z = jnp.zeros((DG, DG), w.dtype)
    return jnp.block([[w[0], z], [z, w[1]]])


def _assemble_f32(bytes_f32):
    b = bytes_f32.astype(jnp.int32)
    bits = (b[:, 0 * DG:1 * DG]
            | (b[:, 1 * DG:2 * DG] << 8)
            | (b[:, 2 * DG:3 * DG] << 16)
            | (b[:, 3 * DG:4 * DG] << 24))
    return jax.lax.bitcast_convert_type(bits, jnp.float32)


def _vq_body(x_ref, Winbd_ref, bin_ref, Woutbd_ref, bout_ref,
             cbTbd_ref, pcat_ref, cbn_ref,
             quant_ref, recon_ref, dsum_ref):
    x = x_ref[...]
    iota = jax.lax.broadcasted_iota(jnp.int32, (TB, CS), 1)

    xin = jnp.dot(_bf(x), _bf(Winbd_ref[...]),
                  preferred_element_type=jnp.float32) + bin_ref[...][None, :]
    r = xin

    dsum = jnp.zeros((), jnp.float32)
    for q in range(NQ):
        ab = jnp.dot(_bf(r), cbTbd_ref[q],
                     preferred_element_type=jnp.float32)
        quants = []
        for g in range(G):
            d = ab[:, g * CS:(g + 1) * CS] \
                + cbn_ref[q, g * CS:(g + 1) * CS][None, :]
            idx = jnp.argmin(d, axis=-1)
            oh = (iota == idx[:, None]).astype(jnp.bfloat16)
            bytes_f32 = jnp.dot(oh, pcat_ref[g, q],
                                preferred_element_type=jnp.float32)
            quants.append(_assemble_f32(bytes_f32))
        quant = jnp.concatenate(quants, axis=-1)
        r = r - quant
        dsum = dsum + jnp.sum(r * r)

    qout = xin - r
    quantized = jnp.dot(_bf(qout), _bf(Woutbd_ref[...]),
                        preferred_element_type=jnp.float32) \
        + bout_ref[...][None, :]
    quant_ref[...] = quantized
    diff = x - quantized
    recon_ref[...] = jnp.sum(diff * diff, axis=-1) * (1.0 / D)
    dsum_ref[...] = dsum.reshape(1, 1, 1)


def _loss_body(dsum_ref, recon_ref, loss_ref, cm_ref):
    cm = jnp.sum(dsum_ref[...]) * (1.0 / (G * NQ * BT * DG))
    loss_ref[...] = recon_ref[...] + cm
    cm_ref[...] = cm.reshape(1, 1)


def _pipeline(x, W_in, b_in, W_out, b_out, codebooks, interpret=False):
    Bb, Tt, _ = x.shape
    xf = x.reshape(BT, D)

    cbT = (codebooks * -2.0).transpose(0, 1, 3, 2).astype(jnp.bfloat16)
    zpad = jnp.zeros((NQ, DG, CS), jnp.bfloat16)
    top = jnp.concatenate([cbT[0], zpad], axis=2)
    bot = jnp.concatenate([zpad, cbT[1]], axis=2)
    cbTbd = jnp.concatenate([top, bot], axis=1)
    pcat = _byte_planes_cat(codebooks)
    cbn = jnp.sum(codebooks * codebooks, axis=-1)
    cbn_cat = jnp.concatenate([cbn[0], cbn[1]], axis=-1)
    Winbd = _block_diag2(W_in)
    Woutbd = _block_diag2(W_out)
    bin_cat = b_in.reshape(D)
    bout_cat = b_out.reshape(D)

    quantized, recon, dsum = pl.pallas_call(
        _vq_body,
        grid=(NB,),
        in_specs=[
            pl.BlockSpec((TB, D), lambda i: (i, 0)),
            _full((D, D)),
            _full((D,)),
            _full((D, D)),
            _full((D,)),
            _full((NQ, D, 2 * CS)),
            _full((G, NQ, CS, 4 * DG)),
            _full((NQ, 2 * CS)),
        ],
        out_specs=[
            pl.BlockSpec((TB, D), lambda i: (i, 0)),
            pl.BlockSpec((TB,), lambda i: (i,)),
            pl.BlockSpec((1, 1, 1), lambda i: (i, 0, 0)),
        ],
        out_shape=[
            jax.ShapeDtypeStruct((BT, D), jnp.float32),
            jax.ShapeDtypeStruct((BT,), jnp.float32),
            jax.ShapeDtypeStruct((NB, 1, 1), jnp.float32),
        ],
        compiler_params=pltpu.CompilerParams(
            dimension_semantics=("parallel",)),
        interpret=interpret,
    )(xf, Winbd, bin_cat, Woutbd, bout_cat, cbTbd, pcat, cbn_cat)

    loss, cm = pl.pallas_call(
        _loss_body,
        grid=(NB,),
        in_specs=[
            _full((NB, 1, 1)),
            pl.BlockSpec((TB,), lambda i: (i,)),
        ],
        out_specs=[
            pl.BlockSpec((TB,), lambda i: (i,)),
            pl.BlockSpec((1, 1), lambda i: (0, 0)),
        ],
        out_shape=[
            jax.ShapeDtypeStruct((BT,), jnp.float32),
            jax.ShapeDtypeStruct((1, 1), jnp.float32),
        ],
        compiler_params=pltpu.CompilerParams(
            dimension_semantics=("arbitrary",)),
        interpret=interpret,
    )(dsum, recon)

    return (quantized.reshape(Bb, Tt, D), loss.reshape(Bb, Tt),
            cm.reshape(()), recon.reshape(Bb, Tt))


def kernel(x, W_in, b_in, W_out, b_out, codebooks):
    return _pipeline(x, W_in, b_in, W_out, b_out, codebooks)


# speedup vs baseline: 3.5834x; 1.0029x over previous
import jax
import jax.numpy as jnp
from jax.experimental import pallas as pl
from jax.experimental.pallas import tpu as pltpu

G = 2
NQ = 4
CS = 4096
D = 128
DG = D // G
BT = 16 * 2048
TB = 512
NB = BT // TB


def _bf(v):
    return v.astype(jnp.bfloat16)


def _full(spec):
    return pl.BlockSpec(spec, lambda *_: tuple(0 for _ in spec))


def _planes_cat(cb):
    def trunc16(v):
        bits = jax.lax.bitcast_convert_type(v, jnp.int32)
        return jax.lax.bitcast_convert_type(
            bits & jnp.int32(-65536), jnp.float32)
    hi = trunc16(cb)
    r1 = cb - hi
    mid = trunc16(r1)
    lo = r1 - mid
    return jnp.concatenate([_bf(hi), _bf(mid), _bf(lo)], axis=-1)


def _block_diag2(w):
    z = jnp.zeros((DG, DG), w.dtype)
    return jnp.block([[w[0], z], [z, w[1]]])


def _assemble_f32(planes_f32):
    return ((planes_f32[:, 0 * DG:1 * DG]
             + planes_f32[:, 1 * DG:2 * DG])
            + planes_f32[:, 2 * DG:3 * DG])


def _vq_body(x_ref, Winbd_ref, bin_ref, Woutbd_ref, bout_ref,
             cbTbd_ref, pcat_ref, cbn_ref,
             quant_ref, recon_ref, dsum_ref):
    x = x_ref[...]
    iota = jax.lax.broadcasted_iota(jnp.int32, (TB, CS), 1)

    xin = jnp.dot(_bf(x), _bf(Winbd_ref[...]),
                  preferred_element_type=jnp.float32) + bin_ref[...][None, :]
    r = xin

    dsum = jnp.zeros((), jnp.float32)
    for q in range(NQ):
        ab = jnp.dot(_bf(r), cbTbd_ref[q],
                     preferred_element_type=jnp.float32)
        quants = []
        for g in range(G):
            d = ab[:, g * CS:(g + 1) * CS] \
                + cbn_ref[q, g * CS:(g + 1) * CS][None, :]
            idx = jnp.argmin(d, axis=-1)
            oh = (iota == idx[:, None]).astype(jnp.bfloat16)
            planes_f32 = jnp.dot(oh, pcat_ref[g, q],
                                 preferred_element_type=jnp.float32)
            quants.append(_assemble_f32(planes_f32))
        quant = jnp.concatenate(quants, axis=-1)
        r = r - quant
        dsum = dsum + jnp.sum(r * r)

    qout = xin - r
    quantized = jnp.dot(_bf(qout), _bf(Woutbd_ref[...]),
                        preferred_element_type=jnp.float32) \
        + bout_ref[...][None, :]
    quant_ref[...] = quantized
    diff = x - quantized
    recon_ref[...] = jnp.sum(diff * diff, axis=-1) * (1.0 / D)
    dsum_ref[...] = dsum.reshape(1, 1, 1)


def _loss_body(dsum_ref, recon_ref, loss_ref, cm_ref):
    cm = jnp.sum(dsum_ref[...]) * (1.0 / (G * NQ * BT * DG))
    loss_ref[...] = recon_ref[...] + cm
    cm_ref[...] = cm.reshape(1, 1)


def _pipeline(x, W_in, b_in, W_out, b_out, codebooks, interpret=False):
    Bb, Tt, _ = x.shape
    xf = x.reshape(BT, D)

    cbT = (codebooks * -2.0).transpose(0, 1, 3, 2).astype(jnp.bfloat16)
    zpad = jnp.zeros((NQ, DG, CS), jnp.bfloat16)
    top = jnp.concatenate([cbT[0], zpad], axis=2)
    bot = jnp.concatenate([zpad, cbT[1]], axis=2)
    cbTbd = jnp.concatenate([top, bot], axis=1)
    pcat = _planes_cat(codebooks)
    cbn = jnp.sum(codebooks * codebooks, axis=-1)
    cbn_cat = jnp.concatenate([cbn[0], cbn[1]], axis=-1)
    Winbd = _block_diag2(W_in)
    Woutbd = _block_diag2(W_out)
    bin_cat = b_in.reshape(D)
    bout_cat = b_out.reshape(D)

    quantized, recon, dsum = pl.pallas_call(
        _vq_body,
        grid=(NB,),
        in_specs=[
            pl.BlockSpec((TB, D), lambda i: (i, 0)),
            _full((D, D)),
            _full((D,)),
            _full((D, D)),
            _full((D,)),
            _full((NQ, D, 2 * CS)),
            _full((G, NQ, CS, 3 * DG)),
            _full((NQ, 2 * CS)),
        ],
        out_specs=[
            pl.BlockSpec((TB, D), lambda i: (i, 0)),
            pl.BlockSpec((TB,), lambda i: (i,)),
            pl.BlockSpec((1, 1, 1), lambda i: (i, 0, 0)),
        ],
        out_shape=[
            jax.ShapeDtypeStruct((BT, D), jnp.float32),
            jax.ShapeDtypeStruct((BT,), jnp.float32),
            jax.ShapeDtypeStruct((NB, 1, 1), jnp.float32),
        ],
        compiler_params=pltpu.CompilerParams(
            dimension_semantics=("parallel",)),
        interpret=interpret,
    )(xf, Winbd, bin_cat, Woutbd, bout_cat, cbTbd, pcat, cbn_cat)

    loss, cm = pl.pallas_call(
        _loss_body,
        grid=(NB,),
        in_specs=[
            _full((NB, 1, 1)),
            pl.BlockSpec((TB,), lambda i: (i,)),
        ],
        out_specs=[
            pl.BlockSpec((TB,), lambda i: (i,)),
            pl.BlockSpec((1, 1), lambda i: (0, 0)),
        ],
        out_shape=[
            jax.ShapeDtypeStruct((BT,), jnp.float32),
            jax.ShapeDtypeStruct((1, 1), jnp.float32),
        ],
        compiler_params=pltpu.CompilerParams(
            dimension_semantics=("arbitrary",)),
        interpret=interpret,
    )(dsum, recon)

    return (quantized.reshape(Bb, Tt, D), loss.reshape(Bb, Tt),
            cm.reshape(()), recon.reshape(Bb, Tt))


def kernel(x, W_in, b_in, W_out, b_out, codebooks):
    return _pipeline(x, W_in, b_in, W_out, b_out, codebooks)
